# baseline (speedup 1.0000x reference)
import functools

import jax
import jax.numpy as jnp
from jax import lax
from jax.experimental import pallas as pl
from jax.experimental.pallas import tpu as pltpu
from jax.experimental.pallas import tpu_sc as plsc

_VOCAB = 1000000
_CHUNK = 128


def _make_lookup(batch, vocab, dim, nc, ns):
    num_workers = nc * ns
    b_per_w = batch // num_workers
    n_chunks = b_per_w // _CHUNK
    mesh = plsc.VectorSubcoreMesh(core_axis_name="c", subcore_axis_name="s")

    @functools.partial(
        pl.kernel,
        mesh=mesh,
        out_type=jax.ShapeDtypeStruct((batch, dim), jnp.float32),
        scratch_types=[
            pltpu.VMEM((n_chunks, _CHUNK), jnp.int32),
            pltpu.VMEM((b_per_w, dim), jnp.float32),
            pltpu.SemaphoreType.DMA,
            pltpu.SemaphoreType.DMA,
        ],
        compiler_params=pltpu.CompilerParams(use_tc_tiling_on_sc=False),
    )
    def lookup(x_hbm, table_hbm, out_hbm, idx_v, rows_v, gsem, osem):
        wid = lax.axis_index("s") * nc + lax.axis_index("c")
        base = wid * b_per_w
        pltpu.sync_copy(x_hbm.at[pl.ds(wid * n_chunks, n_chunks)], idx_v)
        for i in range(b_per_w // 16):
            r, c = (i * 16) // _CHUNK, (i * 16) % _CHUNK
            v = idx_v[r, pl.ds(c, 16)]
            idx_v[r, pl.ds(c, 16)] = jnp.minimum(
                jnp.maximum(v, jnp.int32(0)), jnp.int32(vocab - 1)
            )
        for j in range(n_chunks):
            pltpu.async_copy(
                table_hbm.at[idx_v.at[j]],
                rows_v.at[pl.ds(j * _CHUNK, _CHUNK)],
                gsem,
            )
        for j in range(n_chunks):
            pltpu.make_async_copy(
                table_hbm.at[idx_v.at[j]],
                rows_v.at[pl.ds(j * _CHUNK, _CHUNK)],
                gsem,
            ).wait()
        pltpu.async_copy(rows_v, out_hbm.at[pl.ds(base, b_per_w)], osem).wait()

    return lookup


def kernel(x, table):
    vocab, dim = table.shape
    info = plsc.get_sparse_core_info()
    fn = _make_lookup(x.shape[0], vocab, dim, info.num_cores, info.num_subcores)
    x2d = x.astype(jnp.int32).reshape(-1, _CHUNK)
    return fn(x2d, table)
